# baseline (speedup 1.0000x reference)
import jax
import jax.numpy as jnp
from jax import lax
from jax.experimental import pallas as pl
from jax.experimental.pallas import tpu as pltpu
from jax.experimental.pallas import tpu_sc as plsc

N_NODES = 100000
MAX_DEGREE = 64
NUM_SAMPLES = 32
BATCH = 65536

_PERM32 = (35, 45, 31, 63, 7, 4, 29, 44, 16, 58, 37, 19, 61, 2, 34, 5,
           30, 42, 3, 39, 56, 22, 6, 54, 18, 10, 11, 53, 32, 15, 49, 50)

_NC, _NS, _LANES = 2, 16, 16
_NW = _NC * _NS
_CH = 4096
_NCH = BATCH // _CH


def _body(table_hbm, idx_hbm, out_hbm, col_v, idx_v0, idx_v1, out_v0,
          out_v1, isem0, isem1, osem0, osem1):
    w = lax.axis_index("s") * _NC + lax.axis_index("c")
    p = jnp.int32(0)
    for k in range(_NW):
        p = jnp.where(w == k, jnp.int32(_PERM32[k]), p)
    pltpu.sync_copy(table_hbm.at[pl.ds(p, 1), :], col_v)

    isems = (isem0, isem1)
    osems = (osem0, osem1)
    idx_vs = (idx_v0, idx_v1)
    out_vs = (out_v0, out_v1)
    pltpu.async_copy(idx_hbm.at[pl.ds(0, _CH)], idx_v0, isems[0])
    pltpu.async_copy(idx_hbm.at[pl.ds(_CH, _CH)], idx_v1, isems[1])

    def super_body(s, _):
        for b in range(2):
            g = s * 2 + b
            pltpu.make_async_copy(
                idx_hbm.at[pl.ds(0, _CH)], idx_vs[b], isems[b]).wait()

            @pl.when(g >= 2)
            def _():
                pltpu.make_async_copy(
                    out_vs[b], out_hbm.at[pl.ds(0, 1), pl.ds(0, _CH)],
                    osems[b]).wait()

            idx_b = idx_vs[b]
            out_b = out_vs[b]
            zeros16 = jnp.zeros((_LANES,), dtype=jnp.int32)

            def vec_body(i, _):
                base = i * (8 * _LANES)
                vs = [idx_b[pl.ds(base + k * _LANES, _LANES)]
                      for k in range(8)]
                gs = [plsc.load_gather(col_v, [zeros16, v]) for v in vs]
                for k in range(8):
                    out_b[0, pl.ds(base + k * _LANES, _LANES)] = gs[k]
                return 0

            lax.fori_loop(0, _CH // (8 * _LANES), vec_body, 0)
            pltpu.async_copy(
                out_vs[b], out_hbm.at[pl.ds(w, 1), pl.ds(g * _CH, _CH)],
                osems[b])

            @pl.when(g + 2 < _NCH)
            def _():
                pltpu.async_copy(
                    idx_hbm.at[pl.ds((g + 2) * _CH, _CH)], idx_vs[b],
                    isems[b])
        return 0

    lax.fori_loop(0, _NCH // 2, super_body, 0)
    for b in range(2):
        pltpu.make_async_copy(
            out_vs[b], out_hbm.at[pl.ds(0, 1), pl.ds(0, _CH)],
            osems[b]).wait()


@jax.jit
def kernel(inputs, adj_info):
    mesh = plsc.VectorSubcoreMesh(core_axis_name="c", subcore_axis_name="s")
    out_t = pl.kernel(
        _body,
        out_type=jax.ShapeDtypeStruct((NUM_SAMPLES, BATCH), jnp.int32),
        mesh=mesh,
        compiler_params=pltpu.CompilerParams(needs_layout_passes=False),
        scratch_types=[
            pltpu.VMEM((1, N_NODES), jnp.int32),
            pltpu.VMEM((_CH,), jnp.int32),
            pltpu.VMEM((_CH,), jnp.int32),
            pltpu.VMEM((1, _CH), jnp.int32),
            pltpu.VMEM((1, _CH), jnp.int32),
            pltpu.SemaphoreType.DMA,
            pltpu.SemaphoreType.DMA,
            pltpu.SemaphoreType.DMA,
            pltpu.SemaphoreType.DMA,
        ],
    )(adj_info.T, inputs)
    return out_t.T


# speedup vs baseline: 1.0072x; 1.0072x over previous
import jax
import jax.numpy as jnp
from jax import lax
from jax.experimental import pallas as pl
from jax.experimental.pallas import tpu as pltpu
from jax.experimental.pallas import tpu_sc as plsc

N_NODES = 100000
MAX_DEGREE = 64
NUM_SAMPLES = 32
BATCH = 65536

_PERM32 = (35, 45, 31, 63, 7, 4, 29, 44, 16, 58, 37, 19, 61, 2, 34, 5,
           30, 42, 3, 39, 56, 22, 6, 54, 18, 10, 11, 53, 32, 15, 49, 50)

_NC, _NS, _LANES = 2, 16, 16
_NW = _NC * _NS
_CH = 4096
_NCH = BATCH // _CH


def _body(table_hbm, idx_hbm, out_hbm, col_v, idx_v0, idx_v1, out_v0,
          out_v1, isem0, isem1, osem0, osem1):
    w = lax.axis_index("s") * _NC + lax.axis_index("c")
    p = jnp.int32(0)
    for k in range(_NW):
        p = jnp.where(w == k, jnp.int32(_PERM32[k]), p)
    pltpu.sync_copy(table_hbm.at[pl.ds(p, 1), :], col_v)

    isems = (isem0, isem1)
    osems = (osem0, osem1)
    idx_vs = (idx_v0, idx_v1)
    out_vs = (out_v0, out_v1)
    pltpu.async_copy(idx_hbm.at[pl.ds(0, _CH)], idx_v0, isems[0])
    pltpu.async_copy(idx_hbm.at[pl.ds(_CH, _CH)], idx_v1, isems[1])

    def super_body(s, _):
        for b in range(2):
            g = s * 2 + b
            pltpu.make_async_copy(
                idx_hbm.at[pl.ds(0, _CH)], idx_vs[b], isems[b]).wait()

            @pl.when(g >= 2)
            def _():
                pltpu.make_async_copy(
                    out_vs[b], out_hbm.at[pl.ds(0, 1), pl.ds(0, _CH)],
                    osems[b]).wait()

            idx_b = idx_vs[b]
            out_b = out_vs[b]
            zeros16 = jnp.zeros((_LANES,), dtype=jnp.int32)

            def vec_body(i, _):
                base = i * (8 * _LANES)
                vs = [idx_b[pl.ds(base + k * _LANES, _LANES)]
                      for k in range(8)]
                gs = [plsc.load_gather(col_v, [zeros16, v]) for v in vs]
                for k in range(8):
                    out_b[0, pl.ds(base + k * _LANES, _LANES)] = gs[k]
                return 0

            lax.fori_loop(0, _CH // (8 * _LANES), vec_body, 0)
            pltpu.async_copy(
                out_vs[b], out_hbm.at[pl.ds(w, 1), pl.ds(g * _CH, _CH)],
                osems[b])

            @pl.when(g + 2 < _NCH)
            def _():
                pltpu.async_copy(
                    idx_hbm.at[pl.ds((g + 2) * _CH, _CH)], idx_vs[b],
                    isems[b])
        return 0

    lax.fori_loop(0, _NCH // 2, super_body, 0)
    for b in range(2):
        pltpu.make_async_copy(
            out_vs[b], out_hbm.at[pl.ds(0, 1), pl.ds(0, _CH)],
            osems[b]).wait()


@jax.jit
def kernel(inputs, adj_info):
    mesh = plsc.VectorSubcoreMesh(core_axis_name="c", subcore_axis_name="s")
    out_t = pl.kernel(
        _body,
        out_type=jax.ShapeDtypeStruct((NUM_SAMPLES, BATCH), jnp.int32),
        mesh=mesh,
        compiler_params=pltpu.CompilerParams(
            needs_layout_passes=False,
            disable_bounds_checks=True,
            disable_semaphore_checks=True,
            skip_device_barrier=True,
        ),
        scratch_types=[
            pltpu.VMEM((1, N_NODES), jnp.int32),
            pltpu.VMEM((_CH,), jnp.int32),
            pltpu.VMEM((_CH,), jnp.int32),
            pltpu.VMEM((1, _CH), jnp.int32),
            pltpu.VMEM((1, _CH), jnp.int32),
            pltpu.SemaphoreType.DMA,
            pltpu.SemaphoreType.DMA,
            pltpu.SemaphoreType.DMA,
            pltpu.SemaphoreType.DMA,
        ],
    )(adj_info.T, inputs)
    return out_t.T


# speedup vs baseline: 1.4341x; 1.4238x over previous
import jax
import jax.numpy as jnp
from jax import lax
from jax.experimental import pallas as pl
from jax.experimental.pallas import tpu as pltpu
from jax.experimental.pallas import tpu_sc as plsc

N_NODES = 100000
MAX_DEGREE = 64
NUM_SAMPLES = 32
BATCH = 65536

_PERM32 = (35, 45, 31, 63, 7, 4, 29, 44, 16, 58, 37, 19, 61, 2, 34, 5,
           30, 42, 3, 39, 56, 22, 6, 54, 18, 10, 11, 53, 32, 15, 49, 50)

_NC, _NS, _LANES = 2, 16, 16
_NW = _NC * _NS
_CH = 4096
_NCH = BATCH // _CH


def _body(table_hbm, idx_hbm, out_hbm, col_v, sh_idx, idx_v0, idx_v1,
          out_v0, out_v1, csem, isem0, isem1, osem0, osem1):
    s_ax = lax.axis_index("s")
    w = s_ax * _NC + lax.axis_index("c")
    p = jnp.int32(0)
    for k in range(_NW):
        p = jnp.where(w == k, jnp.int32(_PERM32[k]), p)
    pltpu.async_copy(table_hbm.at[pl.ds(p, 1), :], col_v, csem)

    @pl.when(s_ax == 0)
    def _():
        pltpu.sync_copy(idx_hbm, sh_idx)

    plsc.subcore_barrier()

    isems = (isem0, isem1)
    osems = (osem0, osem1)
    idx_vs = (idx_v0, idx_v1)
    out_vs = (out_v0, out_v1)
    pltpu.async_copy(sh_idx.at[pl.ds(0, _CH)], idx_v0, isems[0])
    pltpu.async_copy(sh_idx.at[pl.ds(_CH, _CH)], idx_v1, isems[1])
    pltpu.make_async_copy(table_hbm.at[pl.ds(p, 1), :], col_v, csem).wait()

    def super_body(s, _):
        for b in range(2):
            g = s * 2 + b
            pltpu.make_async_copy(
                sh_idx.at[pl.ds(0, _CH)], idx_vs[b], isems[b]).wait()

            @pl.when(g >= 2)
            def _():
                pltpu.make_async_copy(
                    out_vs[b], out_hbm.at[pl.ds(0, 1), pl.ds(0, _CH)],
                    osems[b]).wait()

            idx_b = idx_vs[b]
            out_b = out_vs[b]
            zeros16 = jnp.zeros((_LANES,), dtype=jnp.int32)

            def vec_body(i, _):
                base = i * (8 * _LANES)
                vs = [idx_b[pl.ds(base + k * _LANES, _LANES)]
                      for k in range(8)]
                gs = [plsc.load_gather(col_v, [zeros16, v]) for v in vs]
                for k in range(8):
                    out_b[0, pl.ds(base + k * _LANES, _LANES)] = gs[k]
                return 0

            lax.fori_loop(0, _CH // (8 * _LANES), vec_body, 0)
            pltpu.async_copy(
                out_vs[b], out_hbm.at[pl.ds(w, 1), pl.ds(g * _CH, _CH)],
                osems[b])

            @pl.when(g + 2 < _NCH)
            def _():
                pltpu.async_copy(
                    sh_idx.at[pl.ds((g + 2) * _CH, _CH)], idx_vs[b],
                    isems[b])
        return 0

    lax.fori_loop(0, _NCH // 2, super_body, 0)
    for b in range(2):
        pltpu.make_async_copy(
            out_vs[b], out_hbm.at[pl.ds(0, 1), pl.ds(0, _CH)],
            osems[b]).wait()


@jax.jit
def kernel(inputs, adj_info):
    mesh = plsc.VectorSubcoreMesh(core_axis_name="c", subcore_axis_name="s")
    out_t = pl.kernel(
        _body,
        out_type=jax.ShapeDtypeStruct((NUM_SAMPLES, BATCH), jnp.int32),
        mesh=mesh,
        compiler_params=pltpu.CompilerParams(
            needs_layout_passes=False,
            disable_bounds_checks=True,
            disable_semaphore_checks=True,
            skip_device_barrier=True,
        ),
        scratch_types=[
            pltpu.VMEM((1, N_NODES), jnp.int32),
            pltpu.VMEM_SHARED((BATCH,), jnp.int32),
            pltpu.VMEM((_CH,), jnp.int32),
            pltpu.VMEM((_CH,), jnp.int32),
            pltpu.VMEM((1, _CH), jnp.int32),
            pltpu.VMEM((1, _CH), jnp.int32),
            pltpu.SemaphoreType.DMA,
            pltpu.SemaphoreType.DMA,
            pltpu.SemaphoreType.DMA,
            pltpu.SemaphoreType.DMA,
            pltpu.SemaphoreType.DMA,
        ],
    )(adj_info.T, inputs)
    return out_t.T
